# baseline (speedup 1.0000x reference)
import functools

import jax
import jax.numpy as jnp
from jax.experimental import pallas as pl
from jax.experimental.pallas import tpu as pltpu

CH = 200


def _fused_kernel(sub_ref, adj_ref, x_ref, wcat_ref, bsum_ref, w2_ref,
                  b2_ref, o_ref, buf, s_ref, h_ref, g_ref, sem, *, n):
    hdim = w2_ref.shape[0]
    nch1 = (3 * n) // CH
    nchb = n // CH
    nch = nch1 + nchb

    def _copy(c, slot):
        @pl.when(c < nch1)
        def _():
            pltpu.make_async_copy(
                sub_ref.at[pl.ds(c * CH, CH), :], buf.at[slot],
                sem.at[slot]).start()

        @pl.when(c >= nch1)
        def _():
            pltpu.make_async_copy(
                adj_ref.at[pl.ds((c - nch1) * CH, CH), :], buf.at[slot],
                sem.at[slot]).start()

    def _wait(slot):
        pltpu.make_async_copy(buf.at[slot], buf.at[slot],
                              sem.at[slot]).wait()

    _copy(0, 0)

    x = x_ref[...]
    for k in range(3):
        s_ref[k] = jnp.dot(x, wcat_ref[:, k * hdim:(k + 1) * hdim],
                           preferred_element_type=jnp.float32)
    h_ref[...] = jnp.broadcast_to(bsum_ref[...], h_ref.shape)

    def body(c, carry):
        slot = jax.lax.rem(c, 2)

        @pl.when(c + 1 < nch)
        def _():
            _copy(c + 1, jax.lax.rem(c + 1, 2))

        _wait(slot)
        chunk = buf[slot]

        @pl.when(c < nch1)
        def _():
            b = c // nchb
            r0 = (c - b * nchb) * CH
            acc = h_ref[pl.ds(r0, CH), :] + jnp.dot(
                chunk, s_ref[b], preferred_element_type=jnp.float32)
            h_ref[pl.ds(r0, CH), :] = acc

            @pl.when(b == 2)
            def _():
                g_ref[pl.ds(r0, CH), :] = jnp.dot(
                    jnp.maximum(acc, 0.0), w2_ref[...],
                    preferred_element_type=jnp.float32)

        @pl.when(c >= nch1)
        def _():
            r0 = (c - nch1) * CH
            z = jnp.dot(chunk, g_ref[...],
                        preferred_element_type=jnp.float32)
            z = z + b2_ref[...]
            m = jnp.max(z, axis=1, keepdims=True)
            e = jnp.exp(z - m)
            lse = m + jnp.log(jnp.sum(e, axis=1, keepdims=True))
            o_ref[pl.ds(r0, CH), :] = z - lse

        return carry

    jax.lax.fori_loop(0, nch, body, 0)


@jax.jit
def kernel(x, adj, sub_adj, W1_1, b1_1, W1_2, b1_2, W1_3, b1_3, W2, b2):
    n, f = x.shape
    h = W1_1.shape[1]
    c = W2.shape[1]

    wcat = jnp.concatenate([W1_1, W1_2, W1_3], axis=1)
    bsum = (b1_1 + b1_2 + b1_3).reshape(1, h)
    b2r = b2.reshape(1, c)
    sub_flat = sub_adj.reshape(3 * n, n)

    out = pl.pallas_call(
        functools.partial(_fused_kernel, n=n),
        grid=(1,),
        in_specs=[
            pl.BlockSpec(memory_space=pl.ANY),
            pl.BlockSpec(memory_space=pl.ANY),
            pl.BlockSpec((n, f), lambda i: (0, 0)),
            pl.BlockSpec((f, 3 * h), lambda i: (0, 0)),
            pl.BlockSpec((1, h), lambda i: (0, 0)),
            pl.BlockSpec((h, c), lambda i: (0, 0)),
            pl.BlockSpec((1, c), lambda i: (0, 0)),
        ],
        out_specs=pl.BlockSpec((n, c), lambda i: (0, 0)),
        out_shape=jax.ShapeDtypeStruct((n, c), jnp.float32),
        scratch_shapes=[
            pltpu.VMEM((2, CH, n), jnp.float32),
            pltpu.VMEM((3, n, h), jnp.float32),
            pltpu.VMEM((n, h), jnp.float32),
            pltpu.VMEM((n, c), jnp.float32),
            pltpu.SemaphoreType.DMA((2,)),
        ],
        compiler_params=pltpu.CompilerParams(
            dimension_semantics=("arbitrary",)),
    )(sub_flat, adj, x, wcat, bsum, W2, b2r)

    return out


# speedup vs baseline: 1.1202x; 1.1202x over previous
import functools

import jax
import jax.numpy as jnp
from jax.experimental import pallas as pl
from jax.experimental.pallas import tpu as pltpu

CH = 200


def _fused_kernel(sub_ref, adj_ref, x_ref, wcat_ref, bsum_ref, w2_ref,
                  b2_ref, o_ref, buf, s_ref, h_ref, g_ref, sem, *, n):
    hdim = w2_ref.shape[0]
    nch1 = (3 * n) // CH
    nchb = n // CH
    nch = nch1 + nchb

    def _copy(c, slot):
        @pl.when(c < nch1)
        def _():
            pltpu.make_async_copy(
                sub_ref.at[pl.ds(c * CH, CH), :], buf.at[slot],
                sem.at[slot]).start()

        @pl.when(c >= nch1)
        def _():
            pltpu.make_async_copy(
                adj_ref.at[pl.ds((c - nch1) * CH, CH), :], buf.at[slot],
                sem.at[slot]).start()

    def _wait(slot):
        pltpu.make_async_copy(buf.at[slot], buf.at[slot],
                              sem.at[slot]).wait()

    _copy(0, 0)

    x = x_ref[...]
    for k in range(3):
        s_ref[k] = jnp.dot(x, wcat_ref[:, k * hdim:(k + 1) * hdim],
                           preferred_element_type=jnp.float32)
    h_ref[...] = jnp.broadcast_to(bsum_ref[...], h_ref.shape)

    def _step_prefetch(c):
        @pl.when(c + 1 < nch)
        def _():
            _copy(c + 1, jax.lax.rem(c + 1, 2))

    for b in range(3):
        def body1(j, carry, b=b):
            c = b * nchb + j
            slot = jax.lax.rem(c, 2)
            _step_prefetch(c)
            _wait(slot)
            r0 = j * CH
            acc = h_ref[pl.ds(r0, CH), :] + jnp.dot(
                buf[slot], s_ref[b], preferred_element_type=jnp.float32)
            h_ref[pl.ds(r0, CH), :] = acc
            if b == 2:
                g_ref[pl.ds(r0, CH), :] = jnp.dot(
                    jnp.maximum(acc, 0.0), w2_ref[...],
                    preferred_element_type=jnp.float32)
            return carry

        jax.lax.fori_loop(0, nchb, body1, 0)

    def body2(j, carry):
        c = nch1 + j
        slot = jax.lax.rem(c, 2)
        _step_prefetch(c)
        _wait(slot)
        z = jnp.dot(buf[slot], g_ref[...],
                    preferred_element_type=jnp.float32)
        z = z + b2_ref[...]
        m = jnp.max(z, axis=1, keepdims=True)
        e = jnp.exp(z - m)
        lse = m + jnp.log(jnp.sum(e, axis=1, keepdims=True))
        o_ref[pl.ds(j * CH, CH), :] = z - lse
        return carry

    jax.lax.fori_loop(0, nchb, body2, 0)


@jax.jit
def kernel(x, adj, sub_adj, W1_1, b1_1, W1_2, b1_2, W1_3, b1_3, W2, b2):
    n, f = x.shape
    h = W1_1.shape[1]
    c = W2.shape[1]

    wcat = jnp.concatenate([W1_1, W1_2, W1_3], axis=1)
    bsum = (b1_1 + b1_2 + b1_3).reshape(1, h)
    b2r = b2.reshape(1, c)
    sub_flat = sub_adj.reshape(3 * n, n)

    out = pl.pallas_call(
        functools.partial(_fused_kernel, n=n),
        grid=(1,),
        in_specs=[
            pl.BlockSpec(memory_space=pl.ANY),
            pl.BlockSpec(memory_space=pl.ANY),
            pl.BlockSpec((n, f), lambda i: (0, 0)),
            pl.BlockSpec((f, 3 * h), lambda i: (0, 0)),
            pl.BlockSpec((1, h), lambda i: (0, 0)),
            pl.BlockSpec((h, c), lambda i: (0, 0)),
            pl.BlockSpec((1, c), lambda i: (0, 0)),
        ],
        out_specs=pl.BlockSpec((n, c), lambda i: (0, 0)),
        out_shape=jax.ShapeDtypeStruct((n, c), jnp.float32),
        scratch_shapes=[
            pltpu.VMEM((2, CH, n), jnp.float32),
            pltpu.VMEM((3, n, h), jnp.float32),
            pltpu.VMEM((n, h), jnp.float32),
            pltpu.VMEM((n, c), jnp.float32),
            pltpu.SemaphoreType.DMA((2,)),
        ],
        compiler_params=pltpu.CompilerParams(
            dimension_semantics=("arbitrary",)),
    )(sub_flat, adj, x, wcat, bsum, W2, b2r)

    return out


# speedup vs baseline: 1.1583x; 1.0340x over previous
import functools

import jax
import jax.numpy as jnp
from jax.experimental import pallas as pl
from jax.experimental.pallas import tpu as pltpu

CH = 200
DEPTH = 3


def _fused_kernel(sub_ref, adj_ref, x_ref, wcat_ref, bsum_ref, w2_ref,
                  b2_ref, o_ref, buf, s_ref, h_ref, g_ref, sem, *, n):
    hdim = w2_ref.shape[0]
    nch1 = (3 * n) // CH
    nchb = n // CH
    nch = nch1 + nchb

    def _copy(c, slot):
        @pl.when(c < nch1)
        def _():
            pltpu.make_async_copy(
                sub_ref.at[pl.ds(c * CH, CH), :], buf.at[slot],
                sem.at[slot]).start()

        @pl.when(c >= nch1)
        def _():
            pltpu.make_async_copy(
                adj_ref.at[pl.ds((c - nch1) * CH, CH), :], buf.at[slot],
                sem.at[slot]).start()

    def _wait(slot):
        pltpu.make_async_copy(buf.at[slot], buf.at[slot],
                              sem.at[slot]).wait()

    for p in range(DEPTH - 1):
        _copy(p, p)

    x = x_ref[...]
    for k in range(3):
        s_ref[k] = jnp.dot(x, wcat_ref[:, k * hdim:(k + 1) * hdim],
                           preferred_element_type=jnp.float32)

    def _step_prefetch(c):
        @pl.when(c + DEPTH - 1 < nch)
        def _():
            _copy(c + DEPTH - 1, jax.lax.rem(c + DEPTH - 1, DEPTH))

    for b in range(3):
        def body1(j, carry, b=b):
            c = b * nchb + j
            slot = jax.lax.rem(c, DEPTH)
            _step_prefetch(c)
            _wait(slot)
            r0 = j * CH
            part = jnp.dot(buf[slot], s_ref[b],
                           preferred_element_type=jnp.float32)
            if b == 0:
                h_ref[pl.ds(r0, CH), :] = part + bsum_ref[...]
            elif b == 1:
                h_ref[pl.ds(r0, CH), :] = h_ref[pl.ds(r0, CH), :] + part
            else:
                acc = h_ref[pl.ds(r0, CH), :] + part
                g_ref[pl.ds(r0, CH), :] = jnp.dot(
                    jnp.maximum(acc, 0.0), w2_ref[...],
                    preferred_element_type=jnp.float32)
            return carry

        jax.lax.fori_loop(0, nchb, body1, 0)

    def body2(j, carry):
        c = nch1 + j
        slot = jax.lax.rem(c, DEPTH)
        _step_prefetch(c)
        _wait(slot)
        z = jnp.dot(buf[slot], g_ref[...],
                    preferred_element_type=jnp.float32)
        z = z + b2_ref[...]
        m = jnp.max(z, axis=1, keepdims=True)
        e = jnp.exp(z - m)
        lse = m + jnp.log(jnp.sum(e, axis=1, keepdims=True))
        o_ref[pl.ds(j * CH, CH), :] = z - lse
        return carry

    jax.lax.fori_loop(0, nchb, body2, 0)


@jax.jit
def kernel(x, adj, sub_adj, W1_1, b1_1, W1_2, b1_2, W1_3, b1_3, W2, b2):
    n, f = x.shape
    h = W1_1.shape[1]
    c = W2.shape[1]

    wcat = jnp.concatenate([W1_1, W1_2, W1_3], axis=1)
    bsum = (b1_1 + b1_2 + b1_3).reshape(1, h)
    b2r = b2.reshape(1, c)
    sub_flat = sub_adj.reshape(3 * n, n)

    out = pl.pallas_call(
        functools.partial(_fused_kernel, n=n),
        grid=(1,),
        in_specs=[
            pl.BlockSpec(memory_space=pl.ANY),
            pl.BlockSpec(memory_space=pl.ANY),
            pl.BlockSpec((n, f), lambda i: (0, 0)),
            pl.BlockSpec((f, 3 * h), lambda i: (0, 0)),
            pl.BlockSpec((1, h), lambda i: (0, 0)),
            pl.BlockSpec((h, c), lambda i: (0, 0)),
            pl.BlockSpec((1, c), lambda i: (0, 0)),
        ],
        out_specs=pl.BlockSpec((n, c), lambda i: (0, 0)),
        out_shape=jax.ShapeDtypeStruct((n, c), jnp.float32),
        scratch_shapes=[
            pltpu.VMEM((DEPTH, CH, n), jnp.float32),
            pltpu.VMEM((3, n, h), jnp.float32),
            pltpu.VMEM((n, h), jnp.float32),
            pltpu.VMEM((n, c), jnp.float32),
            pltpu.SemaphoreType.DMA((DEPTH,)),
        ],
        compiler_params=pltpu.CompilerParams(
            dimension_semantics=("arbitrary",)),
    )(sub_flat, adj, x, wcat, bsum, W2, b2r)

    return out
